# baseline (speedup 1.0000x reference)
import functools

import jax
import jax.numpy as jnp
from jax.experimental import pallas as pl
from jax.experimental.pallas import tpu as pltpu

N = 10000
D_IN = 128
D_OUT = 128
NEG_SLOPE = 0.2
BM = 400


def _support_body(x_ref, w_ref, out_ref):
    out_ref[...] = jnp.dot(
        x_ref[...], w_ref[...], preferred_element_type=jnp.float32
    )


def _agg_body(adj_ref, sup_ref, out_ref):
    acc = jnp.dot(adj_ref[...], sup_ref[...], preferred_element_type=jnp.float32)
    out_ref[...] = jnp.where(acc >= 0, acc, NEG_SLOPE * acc)


@jax.jit
def kernel(input_features, adj, W):
    support = pl.pallas_call(
        _support_body,
        out_shape=jax.ShapeDtypeStruct((N, D_OUT), jnp.float32),
    )(input_features, W)

    out = pl.pallas_call(
        _agg_body,
        grid=(N // BM,),
        in_specs=[
            pl.BlockSpec((BM, N), lambda i: (i, 0)),
            pl.BlockSpec((N, D_OUT), lambda i: (0, 0)),
        ],
        out_specs=pl.BlockSpec((BM, D_OUT), lambda i: (i, 0)),
        out_shape=jax.ShapeDtypeStruct((N, D_OUT), jnp.float32),
        compiler_params=pltpu.CompilerParams(
            dimension_semantics=("arbitrary",),
        ),
    )(adj, support)
    return out


# speedup vs baseline: 1.0006x; 1.0006x over previous
import functools

import jax
import jax.numpy as jnp
from jax.experimental import pallas as pl
from jax.experimental.pallas import tpu as pltpu

N = 10000
D_IN = 128
D_OUT = 128
NEG_SLOPE = 0.2
BM = 400


def _support_body(x_ref, w_ref, out_ref):
    out_ref[...] = jnp.dot(
        x_ref[...], w_ref[...], preferred_element_type=jnp.float32
    )


def _agg_body(adj_ref, sup_ref, out_ref):
    a = adj_ref[...].astype(jnp.bfloat16)
    s = sup_ref[...].astype(jnp.bfloat16)
    acc = jnp.dot(a, s, preferred_element_type=jnp.float32)
    out_ref[...] = jnp.where(acc >= 0, acc, NEG_SLOPE * acc)


@jax.jit
def kernel(input_features, adj, W):
    support = pl.pallas_call(
        _support_body,
        out_shape=jax.ShapeDtypeStruct((N, D_OUT), jnp.float32),
    )(input_features, W)

    out = pl.pallas_call(
        _agg_body,
        grid=(N // BM,),
        in_specs=[
            pl.BlockSpec((BM, N), lambda i: (i, 0)),
            pl.BlockSpec((N, D_OUT), lambda i: (0, 0)),
        ],
        out_specs=pl.BlockSpec((BM, D_OUT), lambda i: (i, 0)),
        out_shape=jax.ShapeDtypeStruct((N, D_OUT), jnp.float32),
        compiler_params=pltpu.CompilerParams(
            dimension_semantics=("arbitrary",),
        ),
    )(adj, support)
    return out


# speedup vs baseline: 1.0433x; 1.0426x over previous
import jax
import jax.numpy as jnp
from jax.experimental import pallas as pl
from jax.experimental.pallas import tpu as pltpu

N = 10000
D_IN = 128
D_OUT = 128
NEG_SLOPE = 0.2
BM = 400


def _fused_body(x_ref, w_ref, adj_ref, out_ref, sup_ref):
    @pl.when(pl.program_id(0) == 0)
    def _():
        sup_ref[...] = jnp.dot(
            x_ref[...], w_ref[...], preferred_element_type=jnp.float32
        )

    acc = jnp.dot(adj_ref[...], sup_ref[...], preferred_element_type=jnp.float32)
    out_ref[...] = jnp.where(acc >= 0, acc, NEG_SLOPE * acc)


@jax.jit
def kernel(input_features, adj, W):
    return pl.pallas_call(
        _fused_body,
        grid=(N // BM,),
        in_specs=[
            pl.BlockSpec((N, D_IN), lambda i: (0, 0)),
            pl.BlockSpec((D_IN, D_OUT), lambda i: (0, 0)),
            pl.BlockSpec((BM, N), lambda i: (i, 0)),
        ],
        out_specs=pl.BlockSpec((BM, D_OUT), lambda i: (i, 0)),
        out_shape=jax.ShapeDtypeStruct((N, D_OUT), jnp.float32),
        scratch_shapes=[pltpu.VMEM((N, D_OUT), jnp.float32)],
        compiler_params=pltpu.CompilerParams(
            dimension_semantics=("arbitrary",),
        ),
    )(input_features, W, adj)
